# baseline (device time: 93060 ns/iter reference)
import jax
import jax.numpy as jnp
from jax import lax
from jax.experimental import pallas as pl
from jax.experimental.pallas import tpu as pltpu

N_Y = 4
B = 8


def kernel(partial, resid, gamma):
    _, m, d = partial.shape
    rows = m // B

    def body(p_ref, r_ref, g_ref, out_ref, obuf, rbuf, lbuf,
             rs_sems, rr_sems, ls_sems, lr_sems):
        my_x = lax.axis_index("x")
        my_y = lax.axis_index("y")
        my_z = lax.axis_index("z")
        is_top = my_y == 0
        is_bot = my_y == N_Y - 1

        def copy(buf, send_sems, recv_sems, b, dy):
            return pltpu.make_async_remote_copy(
                src_ref=buf.at[b],
                dst_ref=buf.at[b],
                send_sem=send_sems.at[b],
                recv_sem=recv_sems.at[b],
                device_id=(my_x, my_y + dy, my_z),
                device_id_type=pl.DeviceIdType.MESH,
            )

        barrier_sem = pltpu.get_barrier_semaphore()

        @pl.when(~is_top)
        def _():
            pl.semaphore_signal(
                barrier_sem, inc=1, device_id=(my_x, my_y - 1, my_z),
                device_id_type=pl.DeviceIdType.MESH)

        @pl.when(~is_bot)
        def _():
            pl.semaphore_signal(
                barrier_sem, inc=1, device_id=(my_x, my_y + 1, my_z),
                device_id_type=pl.DeviceIdType.MESH)

        @pl.when(is_top | is_bot)
        def _():
            pl.semaphore_wait(barrier_sem, 1)

        @pl.when(~(is_top | is_bot))
        def _():
            pl.semaphore_wait(barrier_sem, 2)

        obuf[...] = p_ref[0].astype(jnp.bfloat16).reshape(B, rows, d)

        @pl.when(is_top)
        def _():
            for b in range(B):
                rbuf[b] = obuf[b]
                copy(rbuf, rs_sems, rr_sems, b, 1).start()

        @pl.when(is_bot)
        def _():
            for b in range(B):
                lbuf[b] = obuf[b]
                copy(lbuf, ls_sems, lr_sems, b, -1).start()

        for b in range(B):
            @pl.when(~is_top)
            def _(b=b):
                copy(rbuf, rs_sems, rr_sems, b, -1).wait_recv()
                rbuf[b] = rbuf[b] + obuf[b]

            @pl.when(~is_top & ~is_bot)
            def _(b=b):
                copy(rbuf, rs_sems, rr_sems, b, 1).start()

            @pl.when(~is_bot)
            def _(b=b):
                copy(lbuf, ls_sems, lr_sems, b, 1).wait_recv()
                lbuf[b] = lbuf[b] + obuf[b]

            @pl.when(~is_top & ~is_bot)
            def _(b=b):
                copy(lbuf, ls_sems, lr_sems, b, -1).start()

        y = (rbuf[...].astype(jnp.float32).reshape(m, d)
             + lbuf[...].astype(jnp.float32).reshape(m, d)
             - obuf[...].astype(jnp.float32).reshape(m, d)
             + r_ref[...])
        rms = jnp.sqrt(jnp.mean(y * y, axis=-1, keepdims=True) + 1e-6)
        out_ref[...] = y / rms * g_ref[...][None, :]

        for b in range(B):
            @pl.when(~is_bot)
            def _(b=b):
                copy(rbuf, rs_sems, rr_sems, b, 1).wait_send()

            @pl.when(~is_top)
            def _(b=b):
                copy(lbuf, ls_sems, lr_sems, b, -1).wait_send()

    return pl.pallas_call(
        body,
        out_shape=jax.ShapeDtypeStruct((m, d), jnp.float32),
        in_specs=[
            pl.BlockSpec(memory_space=pltpu.VMEM),
            pl.BlockSpec(memory_space=pltpu.VMEM),
            pl.BlockSpec(memory_space=pltpu.VMEM),
        ],
        out_specs=pl.BlockSpec(memory_space=pltpu.VMEM),
        scratch_shapes=[
            pltpu.VMEM((B, rows, d), jnp.bfloat16),
            pltpu.VMEM((B, rows, d), jnp.bfloat16),
            pltpu.VMEM((B, rows, d), jnp.bfloat16),
            pltpu.SemaphoreType.DMA((B,)),
            pltpu.SemaphoreType.DMA((B,)),
            pltpu.SemaphoreType.DMA((B,)),
            pltpu.SemaphoreType.DMA((B,)),
        ],
        compiler_params=pltpu.CompilerParams(collective_id=0),
    )(partial, resid, gamma)


# device time: 48221 ns/iter; 1.9299x vs baseline; 1.9299x over previous
import jax
import jax.numpy as jnp
from jax import lax
from jax.experimental import pallas as pl
from jax.experimental.pallas import tpu as pltpu

N_Y = 4
B = 8
LAG = 3


def kernel(partial, resid, gamma):
    _, m, d = partial.shape
    rows = m // B

    def body(p_ref, r_ref, g_ref, out_ref, obuf, rbuf, lbuf,
             rs_sems, rr_sems, ls_sems, lr_sems):
        my_x = lax.axis_index("x")
        my_y = lax.axis_index("y")
        my_z = lax.axis_index("z")
        is_top = my_y == 0
        is_bot = my_y == N_Y - 1

        def copy(buf, send_sems, recv_sems, b, dy):
            return pltpu.make_async_remote_copy(
                src_ref=buf.at[b],
                dst_ref=buf.at[b],
                send_sem=send_sems.at[b],
                recv_sem=recv_sems.at[b],
                device_id=(my_x, my_y + dy, my_z),
                device_id_type=pl.DeviceIdType.MESH,
            )

        barrier_sem = pltpu.get_barrier_semaphore()

        @pl.when(~is_top)
        def _():
            pl.semaphore_signal(
                barrier_sem, inc=1, device_id=(my_x, my_y - 1, my_z),
                device_id_type=pl.DeviceIdType.MESH)

        @pl.when(~is_bot)
        def _():
            pl.semaphore_signal(
                barrier_sem, inc=1, device_id=(my_x, my_y + 1, my_z),
                device_id_type=pl.DeviceIdType.MESH)

        obuf[...] = p_ref[0].astype(jnp.bfloat16).reshape(B, rows, d)

        @pl.when(is_top | is_bot)
        def _():
            pl.semaphore_wait(barrier_sem, 1)

        @pl.when(~(is_top | is_bot))
        def _():
            pl.semaphore_wait(barrier_sem, 2)

        @pl.when(is_top)
        def _():
            for b in range(B):
                rbuf[b] = obuf[b]
                copy(rbuf, rs_sems, rr_sems, b, 1).start()
            for b in range(B):
                copy(lbuf, ls_sems, lr_sems, b, 1).wait_recv()

        @pl.when(is_bot)
        def _():
            for b in range(B):
                lbuf[b] = obuf[b]
                copy(lbuf, ls_sems, lr_sems, b, -1).start()
            for b in range(B):
                copy(rbuf, rs_sems, rr_sems, b, -1).wait_recv()
                rbuf[b] = rbuf[b] + obuf[b]

        def proc_r(b):
            copy(rbuf, rs_sems, rr_sems, b, -1).wait_recv()
            rbuf[b] = rbuf[b] + obuf[b]
            copy(rbuf, rs_sems, rr_sems, b, 1).start()

        def proc_l(b):
            copy(lbuf, ls_sems, lr_sems, b, 1).wait_recv()
            lbuf[b] = lbuf[b] + obuf[b]
            copy(lbuf, ls_sems, lr_sems, b, -1).start()

        def interleave(lead, lag_):
            for b in range(B + LAG):
                if b < B:
                    lead(b)
                if b >= LAG:
                    lag_(b - LAG)

        @pl.when(my_y == 1)
        def _():
            interleave(proc_r, proc_l)

        @pl.when(my_y == 2)
        def _():
            interleave(proc_l, proc_r)

        @pl.when(is_top)
        def _():
            total = (rbuf[...].astype(jnp.float32)
                     + lbuf[...].astype(jnp.float32))
            out_ref[...] = total.reshape(m, d)

        @pl.when(~is_top)
        def _():
            total = (rbuf[...].astype(jnp.float32)
                     + lbuf[...].astype(jnp.float32)
                     - obuf[...].astype(jnp.float32))
            out_ref[...] = total.reshape(m, d)

        y = out_ref[...] + r_ref[...]
        rms = jnp.sqrt(jnp.mean(y * y, axis=-1, keepdims=True) + 1e-6)
        out_ref[...] = y / rms * g_ref[...][None, :]

        for b in range(B):
            @pl.when(~is_bot)
            def _(b=b):
                copy(rbuf, rs_sems, rr_sems, b, 1).wait_send()

            @pl.when(~is_top)
            def _(b=b):
                copy(lbuf, ls_sems, lr_sems, b, -1).wait_send()

    return pl.pallas_call(
        body,
        out_shape=jax.ShapeDtypeStruct((m, d), jnp.float32),
        in_specs=[
            pl.BlockSpec(memory_space=pltpu.VMEM),
            pl.BlockSpec(memory_space=pltpu.VMEM),
            pl.BlockSpec(memory_space=pltpu.VMEM),
        ],
        out_specs=pl.BlockSpec(memory_space=pltpu.VMEM),
        scratch_shapes=[
            pltpu.VMEM((B, rows, d), jnp.bfloat16),
            pltpu.VMEM((B, rows, d), jnp.bfloat16),
            pltpu.VMEM((B, rows, d), jnp.bfloat16),
            pltpu.SemaphoreType.DMA((B,)),
            pltpu.SemaphoreType.DMA((B,)),
            pltpu.SemaphoreType.DMA((B,)),
            pltpu.SemaphoreType.DMA((B,)),
        ],
        compiler_params=pltpu.CompilerParams(collective_id=0),
    )(partial, resid, gamma)


# device time: 34213 ns/iter; 2.7200x vs baseline; 1.4094x over previous
import jax
import jax.numpy as jnp
from jax import lax
from jax.experimental import pallas as pl
from jax.experimental.pallas import tpu as pltpu

N_Y = 4
B = 8
LAG = 3


def kernel(partial, resid, gamma):
    _, m, d = partial.shape
    half = m // 2
    hr = half // B

    def body(p_ref, r_ref, g_ref, out_ref, obuf, rbuf, lbuf, nout, xin,
             rs_sems, rr_sems, ls_sems, lr_sems, xs_sems, xr_sems):
        my_x = lax.axis_index("x")
        my_y = lax.axis_index("y")
        my_z = lax.axis_index("z")
        is_top = my_y == 0
        is_bot = my_y == N_Y - 1
        h0 = my_x * half
        ph0 = (1 - my_x) * half

        def ycopy(buf, send_sems, recv_sems, b, dy):
            return pltpu.make_async_remote_copy(
                src_ref=buf.at[b],
                dst_ref=buf.at[b],
                send_sem=send_sems.at[b],
                recv_sem=recv_sems.at[b],
                device_id=(my_x, my_y + dy, my_z),
                device_id_type=pl.DeviceIdType.MESH,
            )

        def xcopy(b):
            return pltpu.make_async_remote_copy(
                src_ref=nout.at[b],
                dst_ref=xin.at[b],
                send_sem=xs_sems.at[b],
                recv_sem=xr_sems.at[b],
                device_id=(1 - my_x, my_y, my_z),
                device_id_type=pl.DeviceIdType.MESH,
            )

        barrier_sem = pltpu.get_barrier_semaphore()

        @pl.when(~is_top)
        def _():
            pl.semaphore_signal(
                barrier_sem, inc=1, device_id=(my_x, my_y - 1, my_z),
                device_id_type=pl.DeviceIdType.MESH)

        @pl.when(~is_bot)
        def _():
            pl.semaphore_signal(
                barrier_sem, inc=1, device_id=(my_x, my_y + 1, my_z),
                device_id_type=pl.DeviceIdType.MESH)

        pl.semaphore_signal(
            barrier_sem, inc=1, device_id=(1 - my_x, my_y, my_z),
            device_id_type=pl.DeviceIdType.MESH)

        obuf[...] = p_ref[0, pl.ds(h0, half), :].astype(
            jnp.bfloat16).reshape(B, hr, d)
        g = g_ref[...]

        @pl.when(is_top | is_bot)
        def _():
            pl.semaphore_wait(barrier_sem, 2)

        @pl.when(~(is_top | is_bot))
        def _():
            pl.semaphore_wait(barrier_sem, 3)

        def finalize(b):
            tot = (rbuf[b].astype(jnp.float32)
                   + lbuf[b].astype(jnp.float32)
                   - obuf[b].astype(jnp.float32))
            yv = tot + r_ref[pl.ds(h0 + b * hr, hr), :]
            rms = jnp.sqrt(jnp.mean(yv * yv, axis=-1, keepdims=True) + 1e-6)
            nrm = yv / rms * g[None, :]
            out_ref[pl.ds(h0 + b * hr, hr), :] = nrm
            nout[b] = nrm.astype(jnp.bfloat16)
            xcopy(b).start()

        def proc_r(b):
            ycopy(rbuf, rs_sems, rr_sems, b, -1).wait_recv()
            rbuf[b] = rbuf[b] + obuf[b]

            @pl.when(~is_bot)
            def _():
                ycopy(rbuf, rs_sems, rr_sems, b, 1).start()

        def proc_l(b):
            ycopy(lbuf, ls_sems, lr_sems, b, 1).wait_recv()
            lbuf[b] = lbuf[b] + obuf[b]

            @pl.when(~is_top)
            def _():
                ycopy(lbuf, ls_sems, lr_sems, b, -1).start()

        def interleave(lead, lag_):
            for b in range(B + LAG):
                if b < B:
                    lead(b)
                if b >= LAG:
                    lag_(b - LAG)
                    finalize(b - LAG)

        @pl.when(is_top)
        def _():
            for b in range(B):
                rbuf[b] = obuf[b]
                ycopy(rbuf, rs_sems, rr_sems, b, 1).start()
            for b in range(B):
                proc_l(b)
                finalize(b)

        @pl.when(is_bot)
        def _():
            for b in range(B):
                lbuf[b] = obuf[b]
                ycopy(lbuf, ls_sems, lr_sems, b, -1).start()
            for b in range(B):
                proc_r(b)
                finalize(b)

        @pl.when(my_y == 1)
        def _():
            interleave(proc_r, proc_l)

        @pl.when(my_y == 2)
        def _():
            interleave(proc_l, proc_r)

        for b in range(B):
            xcopy(b).wait_recv()
            out_ref[pl.ds(ph0 + b * hr, hr), :] = xin[b].astype(jnp.float32)

        for b in range(B):
            xcopy(b).wait_send()

            @pl.when(~is_bot)
            def _(b=b):
                ycopy(rbuf, rs_sems, rr_sems, b, 1).wait_send()

            @pl.when(~is_top)
            def _(b=b):
                ycopy(lbuf, ls_sems, lr_sems, b, -1).wait_send()

    return pl.pallas_call(
        body,
        out_shape=jax.ShapeDtypeStruct((m, d), jnp.float32),
        in_specs=[
            pl.BlockSpec(memory_space=pltpu.VMEM),
            pl.BlockSpec(memory_space=pltpu.VMEM),
            pl.BlockSpec(memory_space=pltpu.VMEM),
        ],
        out_specs=pl.BlockSpec(memory_space=pltpu.VMEM),
        scratch_shapes=[
            pltpu.VMEM((B, hr, d), jnp.bfloat16),
            pltpu.VMEM((B, hr, d), jnp.bfloat16),
            pltpu.VMEM((B, hr, d), jnp.bfloat16),
            pltpu.VMEM((B, hr, d), jnp.bfloat16),
            pltpu.VMEM((B, hr, d), jnp.bfloat16),
            pltpu.SemaphoreType.DMA((B,)),
            pltpu.SemaphoreType.DMA((B,)),
            pltpu.SemaphoreType.DMA((B,)),
            pltpu.SemaphoreType.DMA((B,)),
            pltpu.SemaphoreType.DMA((B,)),
            pltpu.SemaphoreType.DMA((B,)),
        ],
        compiler_params=pltpu.CompilerParams(collective_id=0),
    )(partial, resid, gamma)


# device time: 31919 ns/iter; 2.9155x vs baseline; 1.0719x over previous
import jax
import jax.numpy as jnp
from jax import lax
from jax.experimental import pallas as pl
from jax.experimental.pallas import tpu as pltpu

N_Y = 4
B = 16
LAG = 3


def kernel(partial, resid, gamma):
    _, m, d = partial.shape
    half = m // 2
    hr = half // B

    def body(p_ref, r_ref, g_ref, out_ref, obuf, rbuf, lbuf, nout, xin,
             rs_sems, rr_sems, ls_sems, lr_sems, xs_sems, xr_sems):
        my_x = lax.axis_index("x")
        my_y = lax.axis_index("y")
        my_z = lax.axis_index("z")
        is_top = my_y == 0
        is_bot = my_y == N_Y - 1
        h0 = my_x * half
        ph0 = (1 - my_x) * half

        def ycopy(buf, send_sems, recv_sems, b, dy):
            return pltpu.make_async_remote_copy(
                src_ref=buf.at[b],
                dst_ref=buf.at[b],
                send_sem=send_sems.at[b],
                recv_sem=recv_sems.at[b],
                device_id=(my_x, my_y + dy, my_z),
                device_id_type=pl.DeviceIdType.MESH,
            )

        def xcopy(b):
            return pltpu.make_async_remote_copy(
                src_ref=nout.at[b],
                dst_ref=xin.at[b],
                send_sem=xs_sems.at[b],
                recv_sem=xr_sems.at[b],
                device_id=(1 - my_x, my_y, my_z),
                device_id_type=pl.DeviceIdType.MESH,
            )

        barrier_sem = pltpu.get_barrier_semaphore()

        @pl.when(~is_top)
        def _():
            pl.semaphore_signal(
                barrier_sem, inc=1, device_id=(my_x, my_y - 1, my_z),
                device_id_type=pl.DeviceIdType.MESH)

        @pl.when(~is_bot)
        def _():
            pl.semaphore_signal(
                barrier_sem, inc=1, device_id=(my_x, my_y + 1, my_z),
                device_id_type=pl.DeviceIdType.MESH)

        pl.semaphore_signal(
            barrier_sem, inc=1, device_id=(1 - my_x, my_y, my_z),
            device_id_type=pl.DeviceIdType.MESH)

        obuf[...] = p_ref[0, pl.ds(h0, half), :].astype(
            jnp.bfloat16).reshape(B, hr, d)
        g = g_ref[...]

        @pl.when(is_top | is_bot)
        def _():
            pl.semaphore_wait(barrier_sem, 2)

        @pl.when(~(is_top | is_bot))
        def _():
            pl.semaphore_wait(barrier_sem, 3)

        def finalize(b):
            tot = (rbuf[b].astype(jnp.float32)
                   + lbuf[b].astype(jnp.float32)
                   - obuf[b].astype(jnp.float32))
            yv = tot + r_ref[pl.ds(h0 + b * hr, hr), :]
            rinv = lax.rsqrt(jnp.mean(yv * yv, axis=-1, keepdims=True) + 1e-6)
            nrm = yv * rinv * g[None, :]
            nout[b] = nrm.astype(jnp.bfloat16)
            xcopy(b).start()
            out_ref[pl.ds(h0 + b * hr, hr), :] = nrm

        def proc_r(b):
            ycopy(rbuf, rs_sems, rr_sems, b, -1).wait_recv()
            rbuf[b] = rbuf[b] + obuf[b]

            @pl.when(~is_bot)
            def _():
                ycopy(rbuf, rs_sems, rr_sems, b, 1).start()

        def proc_l(b):
            ycopy(lbuf, ls_sems, lr_sems, b, 1).wait_recv()
            lbuf[b] = lbuf[b] + obuf[b]

            @pl.when(~is_top)
            def _():
                ycopy(lbuf, ls_sems, lr_sems, b, -1).start()

        def interleave(lead, lag_):
            for b in range(B + LAG):
                if b < B:
                    lead(b)
                if b >= LAG:
                    lag_(b - LAG)
                    finalize(b - LAG)

        @pl.when(is_top)
        def _():
            for b in range(B):
                rbuf[b] = obuf[b]
                ycopy(rbuf, rs_sems, rr_sems, b, 1).start()
            for b in range(B):
                proc_l(b)
                finalize(b)

        @pl.when(is_bot)
        def _():
            for b in range(B):
                lbuf[b] = obuf[b]
                ycopy(lbuf, ls_sems, lr_sems, b, -1).start()
            for b in range(B):
                proc_r(b)
                finalize(b)

        @pl.when(my_y == 1)
        def _():
            interleave(proc_r, proc_l)

        @pl.when(my_y == 2)
        def _():
            interleave(proc_l, proc_r)

        for b in range(B):
            xcopy(b).wait_recv()
            out_ref[pl.ds(ph0 + b * hr, hr), :] = xin[b].astype(jnp.float32)

        for b in range(B):
            xcopy(b).wait_send()

            @pl.when(~is_bot)
            def _(b=b):
                ycopy(rbuf, rs_sems, rr_sems, b, 1).wait_send()

            @pl.when(~is_top)
            def _(b=b):
                ycopy(lbuf, ls_sems, lr_sems, b, -1).wait_send()

    return pl.pallas_call(
        body,
        out_shape=jax.ShapeDtypeStruct((m, d), jnp.float32),
        in_specs=[
            pl.BlockSpec(memory_space=pltpu.VMEM),
            pl.BlockSpec(memory_space=pltpu.VMEM),
            pl.BlockSpec(memory_space=pltpu.VMEM),
        ],
        out_specs=pl.BlockSpec(memory_space=pltpu.VMEM),
        scratch_shapes=[
            pltpu.VMEM((B, hr, d), jnp.bfloat16),
            pltpu.VMEM((B, hr, d), jnp.bfloat16),
            pltpu.VMEM((B, hr, d), jnp.bfloat16),
            pltpu.VMEM((B, hr, d), jnp.bfloat16),
            pltpu.VMEM((B, hr, d), jnp.bfloat16),
            pltpu.SemaphoreType.DMA((B,)),
            pltpu.SemaphoreType.DMA((B,)),
            pltpu.SemaphoreType.DMA((B,)),
            pltpu.SemaphoreType.DMA((B,)),
            pltpu.SemaphoreType.DMA((B,)),
            pltpu.SemaphoreType.DMA((B,)),
        ],
        compiler_params=pltpu.CompilerParams(collective_id=0),
    )(partial, resid, gamma)
